# baseline (speedup 1.0000x reference)
import functools

import jax
import jax.numpy as jnp
from jax import lax
from jax.experimental import pallas as pl
from jax.experimental.pallas import tpu as pltpu
from jax.experimental.pallas import tpu_sc as plsc

B = 16384
CTX = 6
D = 64
L = 16
NC = 2
NS = 16
NW = NC * NS
RPW = B // NW
C = 128
NCHUNK = RPW // C
OPW = RPW * CTX

_mesh = plsc.VectorSubcoreMesh(
    core_axis_name="c", subcore_axis_name="s", num_cores=NC, num_subcores=NS
)


@functools.partial(
    pl.kernel,
    out_type=jax.ShapeDtypeStruct((B * CTX,), jnp.float32),
    mesh=_mesh,
    scratch_types=[
        pltpu.VMEM((C,), jnp.int32),
        pltpu.VMEM((OPW // C, C), jnp.int32),
        pltpu.VMEM((C, D), jnp.float32),
        pltpu.VMEM((C * CTX, D), jnp.float32),
        pltpu.VMEM((OPW,), jnp.int32),
        pltpu.VMEM((OPW,), jnp.float32),
        pltpu.VMEM((OPW,), jnp.float32),
        pltpu.SemaphoreType.DMA,
    ],
    compiler_params=pltpu.CompilerParams(
        needs_layout_passes=False, use_tc_tiling_on_sc=False
    ),
)
def _w2v_sc(center_hbm, ctx_hbm, mask_hbm, ctable_hbm, xtable_hbm, out_hbm,
            cidx_v, xidx_v, crows_v, xrows_v, mask_v, dots_v, outb_v, sem):
    wid = lax.axis_index("s") * NC + lax.axis_index("c")
    obase = wid * OPW

    pltpu.sync_copy(mask_hbm.at[pl.ds(obase, OPW)], mask_v)
    pltpu.sync_copy(ctx_hbm.at[pl.ds(wid * (OPW // C), OPW // C)], xidx_v)

    for chunk in range(NCHUNK):
        rbase = wid * RPW + chunk * C
        pltpu.sync_copy(center_hbm.at[pl.ds(rbase, C)], cidx_v)

        handles = [pltpu.async_copy(ctable_hbm.at[cidx_v], crows_v, sem)]
        for j in range(CTX):
            handles.append(
                pltpu.async_copy(
                    xtable_hbm.at[xidx_v.at[chunk * CTX + j]],
                    xrows_v.at[pl.ds(j * C, C)],
                    sem,
                )
            )
        for h in handles:
            h.wait()

        dbase = chunk * C * CTX
        lane = lax.iota(jnp.int32, L)
        GROUP = 8
        NVR = GROUP * CTX // L

        def dot_body(g, _):
            res = [jnp.zeros((L,), jnp.float32) for _ in range(NVR)]
            for r in range(GROUP):
                i = g * GROUP + r
                cvecs = [crows_v[i, pl.ds(k * L, L)] for k in range(D // L)]
                for j in range(CTX):
                    flat = i * CTX + j
                    pos = r * CTX + j
                    acc = cvecs[0] * xrows_v[flat, pl.ds(0, L)]
                    for k in range(1, D // L):
                        acc = acc + cvecs[k] * xrows_v[flat, pl.ds(k * L, L)]
                    s = jnp.sum(acc)
                    res[pos // L] = jnp.where(lane == (pos % L), s, res[pos // L])
            for t in range(NVR):
                dots_v[pl.ds(dbase + g * (GROUP * CTX) + t * L, L)] = res[t]
            return 0

        lax.fori_loop(0, C // GROUP, dot_body, 0)

    def epi_body(v, _):
        dvec = dots_v[pl.ds(v * L, L)]
        mvec = mask_v[pl.ds(v * L, L)]
        sig = 1.0 / (1.0 + jnp.exp(-dvec))
        outb_v[pl.ds(v * L, L)] = jnp.where(mvec == 0, 0.0, sig)
        return 0

    lax.fori_loop(0, OPW // L, epi_body, 0)

    pltpu.sync_copy(outb_v, out_hbm.at[pl.ds(obase, OPW)])


def kernel(center, context, mask, center_table, context_table):
    center_flat = center.reshape(-1)
    ctx2d = context.reshape(-1, C)
    mask_flat = mask.reshape(-1)
    out = _w2v_sc(center_flat, ctx2d, mask_flat, center_table, context_table)
    return out.reshape(B, CTX)
